# baseline (speedup 1.0000x reference)
import functools

import jax
import jax.numpy as jnp
from jax import lax
from jax.experimental import pallas as pl
from jax.experimental.pallas import tpu as pltpu
from jax.experimental.pallas import tpu_sc as plsc

_VOCAB = 100000
_D = 64
_H = 64
_B = 1024
_LC = 200
_LQ = 20
_P = 10
_LT = _LC + _LQ
_N = _B * _LT

_NC = 2
_NS = 16
_NW = _NC * _NS
_ROWS_PER_WORKER = _N // _NW
_CHUNK = 88
_NCHUNKS = _ROWS_PER_WORKER // _CHUNK
_NBUF = 8


def _fixup_body(qw_ref, ph_ref, rw_ref, out_ref):
    qw = qw_ref[...]
    ph = ph_ref[...]
    repl = jnp.zeros_like(qw)
    for p in range(_P):
        row = rw_ref[p : p + 1, :]
        repl = jnp.where(ph == p, row, repl)
    out_ref[...] = jnp.where(repl > 0, repl, qw)


def _fixup_t(qw_t, ph_t, rw_t):
    return pl.pallas_call(
        _fixup_body,
        out_shape=jax.ShapeDtypeStruct((_LQ, _B), jnp.int32),
    )(qw_t, ph_t, rw_t)


def _transform_body(wv_ref, pw_ref, gtw_ref, gtb_ref, out_ref):
    e = lax.dot_general(
        wv_ref[...], pw_ref[...],
        dimension_numbers=(((0,), (0,)), ((), ())),
        preferred_element_type=jnp.float32)
    for i in range(2):
        gt = jnp.dot(e, gtw_ref[i], preferred_element_type=jnp.float32)
        gt = gt + gtb_ref[i : i + 1, :]
        g = jax.nn.sigmoid(gt[:, :_H])
        t = jax.nn.relu(gt[:, _H:])
        e = g * t + (1.0 - g) * e
    out_ref[...] = jnp.concatenate([e, jnp.zeros_like(e)], axis=1)


_TROWS = 4096


def _transform_table(word_vectors, proj_w, gw, gb, tw, tb):
    gtw = jnp.concatenate([gw, tw], axis=2)
    gtb = jnp.concatenate([gb, tb], axis=1)
    grid = -(-_VOCAB // _TROWS)
    full = lambda *shape: pl.BlockSpec(shape, lambda i: (0,) * len(shape))
    return pl.pallas_call(
        _transform_body,
        grid=(grid,),
        in_specs=[
            pl.BlockSpec((_D, _TROWS), lambda i: (0, i)),
            full(_D, _H),
            full(2, _H, 2 * _H),
            full(2, 2 * _H),
        ],
        out_specs=pl.BlockSpec((_TROWS, 2 * _H), lambda i: (i, 0)),
        out_shape=jax.ShapeDtypeStruct((_VOCAB, 2 * _H), jnp.float32),
    )(word_vectors.T, proj_w, gtw, gtb)


def _sc_gather(table, idx):
    mesh = plsc.VectorSubcoreMesh(core_axis_name="c", subcore_axis_name="s")

    @functools.partial(
        pl.kernel,
        mesh=mesh,
        out_type=jax.ShapeDtypeStruct((_N, 2 * _H), jnp.float32),
        scratch_types=(
            [pltpu.VMEM((_CHUNK,), jnp.int32) for _ in range(_NBUF)]
            + [pltpu.VMEM((_CHUNK, 2 * _H), jnp.float32) for _ in range(_NBUF)]
            + [pltpu.SemaphoreType.DMA for _ in range(2 * _NBUF)]
        ),
    )
    def k(table_hbm, idx_hbm, out_hbm, *scratch):
        idx_v = scratch[:_NBUF]
        rows_v = scratch[_NBUF : 2 * _NBUF]
        gsem = scratch[2 * _NBUF : 3 * _NBUF]
        osem = scratch[3 * _NBUF : 4 * _NBUF]
        wid = lax.axis_index("s") * _NC + lax.axis_index("c")
        base = wid * _ROWS_PER_WORKER

        def start_gather(ci, b):
            off = base + ci * _CHUNK
            pltpu.sync_copy(idx_hbm.at[pl.ds(off, _CHUNK)], idx_v[b])
            pltpu.async_copy(table_hbm.at[idx_v[b]], rows_v[b], gsem[b])

        def wait_gather(b):
            pltpu.make_async_copy(table_hbm.at[idx_v[b]], rows_v[b],
                                  gsem[b]).wait()

        def start_out(ci, b):
            off = base + ci * _CHUNK
            pltpu.async_copy(rows_v[b], out_hbm.at[pl.ds(off, _CHUNK)],
                             osem[b])

        def wait_out(ci, b):
            off = base + ci * _CHUNK
            pltpu.make_async_copy(rows_v[b], out_hbm.at[pl.ds(off, _CHUNK)],
                                  osem[b]).wait()

        for b in range(_NBUF):
            start_gather(b, b)

        @pl.loop(0, _NCHUNKS - _NBUF, step=_NBUF)
        def _(k):
            for b in range(_NBUF):
                wait_gather(b)
                start_out(k + b, b)
            for b in range(_NBUF):
                wait_out(k + b, b)
                start_gather(k + b + _NBUF, b)

        for b in range(_NBUF):
            wait_gather(b)
            start_out(_NCHUNKS - _NBUF + b, b)
        for b in range(_NBUF):
            wait_out(_NCHUNKS - _NBUF + b, b)

    return k(table, idx)


_GT = 4


def _format_body(in_ref, out_ref):
    for g in range(_GT):
        out_ref[g] = in_ref[pl.ds(g * _B, _B), : _H].T


def _format_output(flat):
    return pl.pallas_call(
        _format_body,
        grid=(_LT // _GT,),
        in_specs=[pl.BlockSpec((_GT * _B, 2 * _H), lambda i: (i, 0))],
        out_specs=pl.BlockSpec((_GT, _H, _B), lambda i: (i, 0, 0)),
        out_shape=jax.ShapeDtypeStruct((_LT, _H, _B), jnp.float32),
    )(flat)


def kernel(cw_idxs, qw_idxs, qw_to_phrases, rw_idxs, word_vectors, proj_w,
           hwy_gate_w, hwy_gate_b, hwy_trans_w, hwy_trans_b):
    cw_t = cw_idxs.astype(jnp.int32).T
    qw_t = qw_idxs.astype(jnp.int32).T
    ph_t = qw_to_phrases.astype(jnp.int32).T
    rw_t = rw_idxs.astype(jnp.int32).T

    new_qw_t = _fixup_t(qw_t, ph_t, rw_t)
    table2 = _transform_table(word_vectors, proj_w, hwy_gate_w, hwy_gate_b,
                              hwy_trans_w, hwy_trans_b)
    idx = jnp.concatenate([cw_t, new_qw_t], axis=0).reshape(-1)
    out = _sc_gather(table2, idx)
    return _format_output(out).transpose(2, 0, 1)
